# baseline (speedup 1.0000x reference)
import jax
import jax.numpy as jnp
from jax import lax
from jax.experimental import pallas as pl
from jax.experimental.pallas import tpu as pltpu
from jax.experimental.pallas import tpu_sc as plsc

_B, _L, _D = 4096, 200, 64
_NC, _NS = 2, 16
_NW = _NC * _NS
_BPW = _B // _NW
_LANES = 16
_OFFS = (0, 96)
_LENS = (96, 104)


def _sc_body(left_hbm, lm_hbm, right_hbm, rm_hbm, out_hbm,
             lb0, rb0, lb1, rb1, lb2, rb2, lb3, rb3,
             ob0, ob1, lmb0, rmb0, lmb1, rmb1,
             in0, in1, in2, in3, ou0, ou1, ms0, ms1):
    wid = lax.axis_index("s") * _NC + lax.axis_index("c")
    base = wid * _BPW

    islots = ((lb0, rb0, in0), (lb1, rb1, in1),
              (lb2, rb2, in2), (lb3, rb3, in3))
    oslots = ((ob0, ou0), (ob1, ou1))
    msl = ((lmb0, rmb0, ms0), (lmb1, rmb1, ms1))

    def issue_mask(g, m):
        lmb, rmb, sem = msl[m]
        b = base + g
        pltpu.make_async_copy(lm_hbm.at[b], lmb, sem).start()
        pltpu.make_async_copy(rm_hbm.at[b], rmb, sem).start()

    def wait_mask(m):
        lmb, rmb, sem = msl[m]
        pltpu.make_async_copy(lm_hbm.at[0], lmb, sem).wait()
        pltpu.make_async_copy(rm_hbm.at[0], rmb, sem).wait()

    def issue_in(g, h, s):
        lb, rb, isem = islots[s]
        b = base + g
        off, n = _OFFS[h], _LENS[h]
        pltpu.make_async_copy(left_hbm.at[b, pl.ds(off, n)], lb, isem).start()
        pltpu.make_async_copy(right_hbm.at[b, pl.ds(off, n)], rb, isem).start()

    def wait_in(h, s):
        lb, rb, isem = islots[s]
        n = _LENS[h]
        pltpu.make_async_copy(left_hbm.at[0, pl.ds(0, n)], lb, isem).wait()
        pltpu.make_async_copy(right_hbm.at[0, pl.ds(0, n)], rb, isem).wait()

    def issue_out(g, h):
        ob, osem = oslots[h]
        b = base + g
        off, n = _OFFS[h], _LENS[h]
        pltpu.make_async_copy(ob, out_hbm.at[b, pl.ds(off, n)], osem).start()

    def wait_out(h):
        ob, osem = oslots[h]
        n = _LENS[h]
        pltpu.make_async_copy(ob, out_hbm.at[0, pl.ds(0, n)], osem).wait()

    dnums = lax.GatherDimensionNumbers(
        offset_dims=(), collapsed_slice_dims=(0,), start_index_map=(0,))

    def compute(m, h, s):
        lb, rb, _ = islots[s]
        ob, _ = oslots[h]
        lmb, rmb, _ = msl[m]
        off, n = _OFFS[h], _LENS[h]

        def bcast(vec, lane):
            idxv = jnp.full((_LANES, 1), lane, dtype=jnp.int32)
            return lax.gather(vec, idxv, dnums, slice_sizes=(1,),
                              mode=lax.GatherScatterMode.PROMISE_IN_BOUNDS)

        def do_group(l0):
            lmg = lmb[pl.ds(off + l0, _LANES)]
            rmg = rmb[pl.ds(off + l0, _LANES)]
            for j in range(_LANES):
                lmv = bcast(lmg, j)
                rmv = bcast(rmg, j)
                for k in range(_D // _LANES):
                    col = k * _LANES
                    lv = lb[l0 + j, pl.ds(col, _LANES)]
                    rv = rb[l0 + j, pl.ds(col, _LANES)]
                    ob[l0 + j, pl.ds(col, _LANES)] = (lv - rv * rmv) * lmv

        ngroups = n // _LANES

        @plsc.parallel_loop(0, ngroups * _LANES, step=_LANES, unroll=1)
        def _(l0):
            do_group(l0)

        if n % _LANES:
            do_group(n - _LANES)

    issue_mask(0, 0)
    for h in (0, 1):
        issue_in(0, h, h)
    issue_mask(1, 1)
    for h in (0, 1):
        issue_in(1, h, 2 + h)

    def pair_body(gp2, carry):
        for bi in (0, 1):
            g = 2 * gp2 + bi
            wait_mask(bi)

            for h in (0, 1):
                s = 2 * bi + h

                @pl.when(g > 0)
                def _():
                    wait_out(h)

                wait_in(h, s)
                compute(bi, h, s)
                issue_out(g, h)

                @pl.when(g + 2 < _BPW)
                def _():
                    issue_in(g + 2, h, s)

            @pl.when(g + 2 < _BPW)
            def _():
                issue_mask(g + 2, bi)

        return carry

    lax.fori_loop(0, _BPW // 2, pair_body, 0)
    wait_out(0)
    wait_out(1)


_sc_call = pl.kernel(
    _sc_body,
    out_type=jax.ShapeDtypeStruct((_B, _L, _D), jnp.float32),
    mesh=plsc.VectorSubcoreMesh(core_axis_name="c", subcore_axis_name="s"),
    scratch_types=[pltpu.VMEM((_LENS[0], _D), jnp.float32)] * 2
    + [pltpu.VMEM((_LENS[1], _D), jnp.float32)] * 2
    + [pltpu.VMEM((_LENS[0], _D), jnp.float32)] * 2
    + [pltpu.VMEM((_LENS[1], _D), jnp.float32)] * 2
    + [pltpu.VMEM((_LENS[0], _D), jnp.float32)]
    + [pltpu.VMEM((_LENS[1], _D), jnp.float32)]
    + [pltpu.VMEM((_L,), jnp.float32)] * 4
    + [pltpu.SemaphoreType.DMA] * 8,
)


def kernel(left, left_mask, right, right_mask):
    lmf = left_mask.astype(jnp.float32)
    rmf = right_mask.astype(jnp.float32)
    return _sc_call(left, lmf, right, rmf)


# speedup vs baseline: 1.0439x; 1.0439x over previous
import jax
import jax.numpy as jnp
from jax import lax
from jax.experimental import pallas as pl
from jax.experimental.pallas import tpu as pltpu
from jax.experimental.pallas import tpu_sc as plsc

_B, _L, _D = 4096, 200, 64
_NC, _NS = 2, 16
_NW = _NC * _NS
_BPW = _B // _NW
_LANES = 16
_OFFS = (0, 48, 96, 144)
_LENS = (48, 48, 48, 56)
_CMAX = 56


def _sc_body(left_hbm, lm_hbm, right_hbm, rm_hbm, out_hbm,
             lb0, rb0, ob0, lb1, rb1, ob1,
             lb2, rb2, ob2, lb3, rb3, ob3,
             lmb0, rmb0, lmb1, rmb1,
             in0, in1, in2, in3, ou0, ou1, ou2, ou3, ms0, ms1):
    wid = lax.axis_index("s") * _NC + lax.axis_index("c")
    base = wid * _BPW

    slots = ((lb0, rb0, ob0, in0, ou0),
             (lb1, rb1, ob1, in1, ou1),
             (lb2, rb2, ob2, in2, ou2),
             (lb3, rb3, ob3, in3, ou3))
    msl = ((lmb0, rmb0, ms0), (lmb1, rmb1, ms1))

    def issue_mask(g, m):
        lmb, rmb, sem = msl[m]
        b = base + g
        pltpu.make_async_copy(lm_hbm.at[b], lmb, sem).start()
        pltpu.make_async_copy(rm_hbm.at[b], rmb, sem).start()

    def wait_mask(m):
        lmb, rmb, sem = msl[m]
        pltpu.make_async_copy(lm_hbm.at[0], lmb, sem).wait()
        pltpu.make_async_copy(rm_hbm.at[0], rmb, sem).wait()

    def issue_in(g, c):
        lb, rb, _, isem, _ = slots[c]
        b = base + g
        off, n = _OFFS[c], _LENS[c]
        pltpu.make_async_copy(left_hbm.at[b, pl.ds(off, n)],
                              lb.at[pl.ds(0, n)], isem).start()
        pltpu.make_async_copy(right_hbm.at[b, pl.ds(off, n)],
                              rb.at[pl.ds(0, n)], isem).start()

    def wait_in(c):
        lb, rb, _, isem, _ = slots[c]
        n = _LENS[c]
        pltpu.make_async_copy(left_hbm.at[0, pl.ds(0, n)],
                              lb.at[pl.ds(0, n)], isem).wait()
        pltpu.make_async_copy(right_hbm.at[0, pl.ds(0, n)],
                              rb.at[pl.ds(0, n)], isem).wait()

    def issue_out(g, c):
        _, _, ob, _, osem = slots[c]
        b = base + g
        off, n = _OFFS[c], _LENS[c]
        pltpu.make_async_copy(ob.at[pl.ds(0, n)],
                              out_hbm.at[b, pl.ds(off, n)], osem).start()

    def wait_out(c):
        _, _, ob, _, osem = slots[c]
        n = _LENS[c]
        pltpu.make_async_copy(ob.at[pl.ds(0, n)],
                              out_hbm.at[0, pl.ds(0, n)], osem).wait()

    dnums = lax.GatherDimensionNumbers(
        offset_dims=(), collapsed_slice_dims=(0,), start_index_map=(0,))

    def compute(m, c):
        lb, rb, ob, _, _ = slots[c]
        lmb, rmb, _ = msl[m]
        off, n = _OFFS[c], _LENS[c]

        def bcast(vec, lane):
            idxv = jnp.full((_LANES, 1), lane, dtype=jnp.int32)
            return lax.gather(vec, idxv, dnums, slice_sizes=(1,),
                              mode=lax.GatherScatterMode.PROMISE_IN_BOUNDS)

        def do_group(l0):
            lmg = lmb[pl.ds(off + l0, _LANES)]
            rmg = rmb[pl.ds(off + l0, _LANES)]
            for j in range(_LANES):
                lmv = bcast(lmg, j)
                rmv = bcast(rmg, j)
                for k in range(_D // _LANES):
                    col = k * _LANES
                    lv = lb[l0 + j, pl.ds(col, _LANES)]
                    rv = rb[l0 + j, pl.ds(col, _LANES)]
                    ob[l0 + j, pl.ds(col, _LANES)] = (lv - rv * rmv) * lmv

        ngroups = n // _LANES

        @plsc.parallel_loop(0, ngroups * _LANES, step=_LANES, unroll=2)
        def _(l0):
            do_group(l0)

        if n % _LANES:
            do_group(n - _LANES)

    issue_mask(0, 0)
    for c in range(4):
        issue_in(0, c)
    issue_mask(1, 1)

    def pair_body(gp2, carry):
        for bi in (0, 1):
            g = 2 * gp2 + bi
            wait_mask(bi)

            for c in range(4):
                @pl.when(g > 0)
                def _():
                    wait_out(c)

                wait_in(c)
                compute(bi, c)
                issue_out(g, c)

                @pl.when(g + 1 < _BPW)
                def _():
                    issue_in(g + 1, c)

            @pl.when(g + 2 < _BPW)
            def _():
                issue_mask(g + 2, bi)

        return carry

    lax.fori_loop(0, _BPW // 2, pair_body, 0)
    for c in range(4):
        wait_out(c)


_sc_call = pl.kernel(
    _sc_body,
    out_type=jax.ShapeDtypeStruct((_B, _L, _D), jnp.float32),
    mesh=plsc.VectorSubcoreMesh(core_axis_name="c", subcore_axis_name="s"),
    scratch_types=[pltpu.VMEM((_CMAX, _D), jnp.float32)] * 12
    + [pltpu.VMEM((_L,), jnp.float32)] * 4
    + [pltpu.SemaphoreType.DMA] * 10,
)


def kernel(left, left_mask, right, right_mask):
    lmf = left_mask.astype(jnp.float32)
    rmf = right_mask.astype(jnp.float32)
    return _sc_call(left, lmf, right, rmf)
